# Initial kernel scaffold; baseline (speedup 1.0000x reference)
import jax
import jax.numpy as jnp
from jax.experimental import pallas as pl


def kernel(x, edge_index, edge_label_index, W1, b1, W2, b2, Wm1, bm1, Wm2, bm2):
    raise NotImplementedError("write your pallas kernel here")


# speedup vs baseline: 1.0139x; 1.0139x over previous
import functools

import jax
import jax.numpy as jnp
from jax.experimental import pallas as pl
from jax.experimental.pallas import tpu as pltpu


def _decode_body(zi_ref, zj_ref, A_ref, B_ref, C_ref, bm1_ref, wm2_ref,
                 o_ref):
    zi = zi_ref[...]
    zj = zj_ref[...]
    u = (jnp.dot(zi, A_ref[...], preferred_element_type=jnp.float32)
         + jnp.dot(zj, B_ref[...], preferred_element_type=jnp.float32)
         + jnp.dot(jnp.abs(zi - zj), C_ref[...],
                   preferred_element_type=jnp.float32)
         + bm1_ref[...])
    u = jnp.maximum(u, 0.0)
    o_ref[...] = jnp.dot(u, wm2_ref[...], preferred_element_type=jnp.float32)


def _decode(zi, zj, Wm1, bm1, Wm2):
    E, H = zi.shape
    BT = 2560
    assert E % BT == 0
    A = Wm1[0:H]
    B = Wm1[H:2 * H]
    C = Wm1[2 * H:3 * H]
    grid = (E // BT,)
    out = pl.pallas_call(
        _decode_body,
        grid=grid,
        in_specs=[
            pl.BlockSpec((BT, H), lambda i: (i, 0)),
            pl.BlockSpec((BT, H), lambda i: (i, 0)),
            pl.BlockSpec((H, H), lambda i: (0, 0)),
            pl.BlockSpec((H, H), lambda i: (0, 0)),
            pl.BlockSpec((H, H), lambda i: (0, 0)),
            pl.BlockSpec((1, H), lambda i: (0, 0)),
            pl.BlockSpec((H, 1), lambda i: (0, 0)),
        ],
        out_specs=pl.BlockSpec((BT, 1), lambda i: (i, 0)),
        out_shape=jax.ShapeDtypeStruct((E, 1), jnp.float32),
    )(zi, zj, A, B, C, bm1.reshape(1, H), Wm2)
    return out


def _gcn_conv(x, edge_index, W, b, n):
    h = x @ W
    loop = jnp.arange(n, dtype=edge_index.dtype)
    src = jnp.concatenate([edge_index[0], loop])
    dst = jnp.concatenate([edge_index[1], loop])
    deg = jnp.zeros((n,), dtype=h.dtype).at[dst].add(1.0)
    dinv = jnp.where(deg > 0, deg ** -0.5, 0.0)
    norm = dinv[src] * dinv[dst]
    msg = h[src] * norm[:, None]
    out = jnp.zeros((n, h.shape[1]), dtype=h.dtype).at[dst].add(msg)
    return out + b


def kernel(x, edge_index, edge_label_index, W1, b1, W2, b2, Wm1, bm1, Wm2,
           bm2):
    n = x.shape[0]
    z = _gcn_conv(x, edge_index, W1, b1, n)
    z = jnp.maximum(z, 0.0)
    z = _gcn_conv(z, edge_index, W2, b2, n)
    zi = z[edge_label_index[0]]
    zj = z[edge_label_index[1]]
    out = _decode(zi, zj, Wm1, bm1, Wm2)
    return (out + bm2).squeeze()


# speedup vs baseline: 6.0528x; 5.9696x over previous
import functools

import jax
import jax.numpy as jnp
from jax import lax
from jax.experimental import pallas as pl
from jax.experimental.pallas import tpu as pltpu
from jax.experimental.pallas import tpu_sc as plsc

NC = 2
NS = 16
NW = NC * NS
LANES = 16

_SC_PARAMS = pltpu.CompilerParams(needs_layout_passes=False)


def _deg_body(dst_ref, hist_hbm, idx_v, hist_v, nvec):
    wid = lax.axis_index("s") * NC + lax.axis_index("c")
    pltpu.sync_copy(dst_ref.at[wid], idx_v)
    zero16 = jnp.zeros((LANES,), jnp.float32)

    def zbody(j, c):
        hist_v[pl.ds(j * LANES, LANES)] = zero16
        return c

    lax.fori_loop(0, hist_v.shape[0] // LANES, zbody, 0)

    lidx = lax.iota(jnp.int32, LANES)
    masks = [lidx == l for l in range(LANES)]
    ones = jnp.ones((LANES,), jnp.float32)

    def ebody(j, c):
        idx = idx_v[j]
        for l in range(LANES):
            plsc.addupdate_scatter(hist_v, [idx], ones, mask=masks[l])
        return c

    lax.fori_loop(0, nvec, ebody, 0)
    pltpu.sync_copy(hist_v, hist_hbm.at[wid])


def _degree_hist(dst, n_nodes):
    e = dst.size
    assert e % (NW * LANES) == 0
    nvec = e // (NW * LANES)
    dst3 = dst.reshape(NW, nvec, LANES)
    mesh = plsc.VectorSubcoreMesh(core_axis_name="c", subcore_axis_name="s")
    k = pl.kernel(
        functools.partial(_deg_body, nvec=nvec),
        out_type=jax.ShapeDtypeStruct((NW, n_nodes), jnp.float32),
        mesh=mesh,
        compiler_params=_SC_PARAMS,
        scratch_types=[
            pltpu.VMEM((nvec, LANES), jnp.int32),
            pltpu.VMEM((n_nodes,), jnp.float32),
        ],
    )
    return k(dst3)


def _scat_body(hs_ref, src_ref, dst_ref, acc_hbm, src_v, dst_v, gbuf, ibuf,
               acc_sp, n_nodes, nchunk, cw):
    c = lax.axis_index("c")
    s = lax.axis_index("s")

    n_init = n_nodes // cw

    def init_one(i, carry):
        ch = s + i * NS
        pltpu.sync_copy(hs_ref.at[c].at[pl.ds(ch * cw, cw)], ibuf)
        pltpu.sync_copy(ibuf, acc_sp.at[pl.ds(ch * cw, cw)])
        return carry

    my_n = (n_init - s + NS - 1) // NS
    lax.fori_loop(0, my_n, init_one, 0)
    plsc.subcore_barrier()

    def chunk_one(j, carry):
        pltpu.sync_copy(src_ref.at[s].at[j], src_v)
        pltpu.sync_copy(dst_ref.at[s].at[j], dst_v)
        pltpu.sync_copy(hs_ref.at[c].at[src_v], gbuf)
        pltpu.sync_copy(gbuf, acc_sp.at[dst_v], add=True)
        return carry

    lax.fori_loop(0, nchunk, chunk_one, 0)
    plsc.subcore_barrier()

    def wb_one(i, carry):
        ch = s + i * NS
        pltpu.sync_copy(acc_sp.at[pl.ds(ch * cw, cw)], ibuf)
        pltpu.sync_copy(ibuf, acc_hbm.at[c].at[pl.ds(ch * cw, cw)])
        return carry

    lax.fori_loop(0, my_n, wb_one, 0)


def _gcn_scatter(hs, src2, dst2):
    _, n, hf = hs.shape
    ns, nchunk, cw = src2.shape
    assert ns == NS and n % cw == 0
    mesh = plsc.VectorSubcoreMesh(core_axis_name="c", subcore_axis_name="s")
    k = pl.kernel(
        functools.partial(_scat_body, n_nodes=n, nchunk=nchunk, cw=cw),
        out_type=jax.ShapeDtypeStruct((2, n, hf), jnp.float32),
        mesh=mesh,
        compiler_params=_SC_PARAMS,
        scratch_types=[
            pltpu.VMEM((cw,), jnp.int32),
            pltpu.VMEM((cw,), jnp.int32),
            pltpu.VMEM((cw, hf), jnp.float32),
            pltpu.VMEM((cw, hf), jnp.float32),
            pltpu.VMEM_SHARED((n, hf), jnp.float32),
        ],
    )
    return k(hs, src2, dst2)


def _pair_body(z_ref, ii_ref, jj_ref, zi_hbm, zj_hbm, ii_v, jj_v, bi, bj,
               nchunk, cw):
    wid = lax.axis_index("s") * NC + lax.axis_index("c")
    pltpu.sync_copy(ii_ref.at[wid], ii_v)
    pltpu.sync_copy(jj_ref.at[wid], jj_v)
    base = wid * nchunk * cw

    def chunk_one(k, carry):
        pltpu.sync_copy(z_ref.at[ii_v.at[k]], bi)
        pltpu.sync_copy(bi, zi_hbm.at[pl.ds(base + k * cw, cw)])
        pltpu.sync_copy(z_ref.at[jj_v.at[k]], bj)
        pltpu.sync_copy(bj, zj_hbm.at[pl.ds(base + k * cw, cw)])
        return carry

    lax.fori_loop(0, nchunk, chunk_one, 0)


def _pair_gather(z, ii3, jj3):
    n, h = z.shape
    nw, nchunk, cw = ii3.shape
    assert nw == NW
    e = nw * nchunk * cw
    mesh = plsc.VectorSubcoreMesh(core_axis_name="c", subcore_axis_name="s")
    k = pl.kernel(
        functools.partial(_pair_body, nchunk=nchunk, cw=cw),
        out_type=[
            jax.ShapeDtypeStruct((e, h), jnp.float32),
            jax.ShapeDtypeStruct((e, h), jnp.float32),
        ],
        mesh=mesh,
        compiler_params=_SC_PARAMS,
        scratch_types=[
            pltpu.VMEM((nchunk, cw), jnp.int32),
            pltpu.VMEM((nchunk, cw), jnp.int32),
            pltpu.VMEM((cw, h), jnp.float32),
            pltpu.VMEM((cw, h), jnp.float32),
        ],
    )
    return k(z, ii3, jj3)


def _t1_body(hist_ref, x_ref, w1_ref, hs_ref):
    deg = jnp.sum(hist_ref[...], axis=0) + 1.0
    dv = deg ** -0.5
    h = jnp.dot(x_ref[...], w1_ref[...], preferred_element_type=jnp.float32)
    hs = h * dv[:, None]
    hf = hs.shape[1] // 2
    hs_ref[0] = hs[:, :hf]
    hs_ref[1] = hs[:, hf:]


def _t1(hist, x, W1):
    n = x.shape[0]
    hid = W1.shape[1]
    return pl.pallas_call(
        _t1_body,
        out_shape=jax.ShapeDtypeStruct((2, n, hid // 2), jnp.float32),
    )(hist, x, W1)


def _t2_body(acc_ref, hist_ref, b1_ref, w2_ref, hs2_ref):
    deg = jnp.sum(hist_ref[...], axis=0) + 1.0
    dv = deg ** -0.5
    a0 = acc_ref[0] * dv[:, None]
    a1 = acc_ref[1] * dv[:, None]
    hf = a0.shape[1]
    z0 = jnp.maximum(a0 + b1_ref[0, :hf], 0.0)
    z1 = jnp.maximum(a1 + b1_ref[0, hf:], 0.0)
    h2 = (jnp.dot(z0, w2_ref[:hf], preferred_element_type=jnp.float32)
          + jnp.dot(z1, w2_ref[hf:], preferred_element_type=jnp.float32))
    hs2 = h2 * dv[:, None]
    hs2_ref[0] = hs2[:, :hf]
    hs2_ref[1] = hs2[:, hf:]


def _t2(acc, hist, b1, W2):
    _, n, hf = acc.shape
    return pl.pallas_call(
        _t2_body,
        out_shape=jax.ShapeDtypeStruct((2, n, hf), jnp.float32),
    )(acc, hist, b1, W2)


def _t3_body(acc_ref, hist_ref, b2_ref, z_ref):
    deg = jnp.sum(hist_ref[...], axis=0) + 1.0
    dv = deg ** -0.5
    hf = acc_ref.shape[2]
    z_ref[:, :hf] = acc_ref[0] * dv[:, None] + b2_ref[0, :hf]
    z_ref[:, hf:] = acc_ref[1] * dv[:, None] + b2_ref[0, hf:]


def _t3(acc, hist, b2):
    _, n, hf = acc.shape
    hid = 2 * hf
    return pl.pallas_call(
        _t3_body,
        out_shape=jax.ShapeDtypeStruct((n, hid), jnp.float32),
    )(acc, hist, b2)


def _t4_body(zi_ref, zj_ref, A_ref, B_ref, C_ref, bm1_ref, wm2_ref, o_ref):
    zi = zi_ref[...]
    zj = zj_ref[...]
    u = (jnp.dot(zi, A_ref[...], preferred_element_type=jnp.float32)
         + jnp.dot(zj, B_ref[...], preferred_element_type=jnp.float32)
         + jnp.dot(jnp.abs(zi - zj), C_ref[...],
                   preferred_element_type=jnp.float32)
         + bm1_ref[...])
    u = jnp.maximum(u, 0.0)
    o_ref[...] = jnp.dot(u, wm2_ref[...], preferred_element_type=jnp.float32)


def _t4(zi, zj, Wm1, bm1, Wm2, bt=2560):
    e, h = zi.shape
    assert e % bt == 0
    A = Wm1[0:h]
    B = Wm1[h:2 * h]
    C = Wm1[2 * h:3 * h]
    return pl.pallas_call(
        _t4_body,
        grid=(e // bt,),
        in_specs=[
            pl.BlockSpec((bt, h), lambda i: (i, 0)),
            pl.BlockSpec((bt, h), lambda i: (i, 0)),
            pl.BlockSpec((h, h), lambda i: (0, 0)),
            pl.BlockSpec((h, h), lambda i: (0, 0)),
            pl.BlockSpec((h, h), lambda i: (0, 0)),
            pl.BlockSpec((1, h), lambda i: (0, 0)),
            pl.BlockSpec((h, 1), lambda i: (0, 0)),
        ],
        out_specs=pl.BlockSpec((bt, 1), lambda i: (i, 0)),
        out_shape=jax.ShapeDtypeStruct((e, 1), jnp.float32),
    )(zi, zj, A, B, C, bm1.reshape(1, h), Wm2)


def kernel(x, edge_index, edge_label_index, W1, b1, W2, b2, Wm1, bm1, Wm2,
           bm2):
    n = x.shape[0]
    e = edge_index.shape[1]
    el = edge_label_index.shape[1]
    cw = 80
    ei = edge_index.astype(jnp.int32)
    eli = edge_label_index.astype(jnp.int32)
    src2 = ei[0].reshape(NS, e // (NS * cw), cw)
    dst2 = ei[1].reshape(NS, e // (NS * cw), cw)
    ii3 = eli[0].reshape(NW, el // (NW * cw), cw)
    jj3 = eli[1].reshape(NW, el // (NW * cw), cw)

    hist = _degree_hist(ei[1], n)
    hs = _t1(hist, x, W1)
    acc = _gcn_scatter(hs, src2, dst2)
    hs2 = _t2(acc, hist, b1.reshape(1, -1), W2)
    acc2 = _gcn_scatter(hs2, src2, dst2)
    z = _t3(acc2, hist, b2.reshape(1, -1))
    zi, zj = _pair_gather(z, ii3, jj3)
    out = _t4(zi, zj, Wm1, bm1, Wm2)
    return (out + bm2).squeeze()


# speedup vs baseline: 9.7139x; 1.6049x over previous
import functools

import jax
import jax.numpy as jnp
from jax import lax
from jax.experimental import pallas as pl
from jax.experimental.pallas import tpu as pltpu
from jax.experimental.pallas import tpu_sc as plsc

NC = 2
NS = 16
NW = NC * NS
LANES = 16

_SC_PARAMS = pltpu.CompilerParams(needs_layout_passes=False)


def _deg_body(dst_ref, hist_hbm, idx_v, hist_v, nvec):
    wid = lax.axis_index("s") * NC + lax.axis_index("c")
    pltpu.sync_copy(dst_ref.at[wid], idx_v)
    zero16 = jnp.zeros((LANES,), jnp.float32)

    def zbody(j, c):
        hist_v[pl.ds(j * LANES, LANES)] = zero16
        return c

    lax.fori_loop(0, hist_v.shape[0] // LANES, zbody, 0)

    lidx = lax.iota(jnp.int32, LANES)
    masks = [lidx == l for l in range(LANES)]
    ones = jnp.ones((LANES,), jnp.float32)

    def ebody(j, c):
        idx = idx_v[j]
        for l in range(LANES):
            plsc.addupdate_scatter(hist_v, [idx], ones, mask=masks[l])
        return c

    lax.fori_loop(0, nvec, ebody, 0)
    pltpu.sync_copy(hist_v, hist_hbm.at[wid])


def _degree_hist(dst, n_nodes):
    e = dst.size
    assert e % (NW * LANES) == 0
    nvec = e // (NW * LANES)
    dst3 = dst.reshape(NW, nvec, LANES)
    mesh = plsc.VectorSubcoreMesh(core_axis_name="c", subcore_axis_name="s")
    k = pl.kernel(
        functools.partial(_deg_body, nvec=nvec),
        out_type=jax.ShapeDtypeStruct((NW, n_nodes), jnp.float32),
        mesh=mesh,
        compiler_params=_SC_PARAMS,
        scratch_types=[
            pltpu.VMEM((nvec, LANES), jnp.int32),
            pltpu.VMEM((n_nodes,), jnp.float32),
        ],
    )
    return k(dst3)


def _scat_body(hs_ref, idx_ref, acc_hbm, ib, gb0, gb1, gsem, ssem,
               acc_sp, n_nodes, nchunk, cw, half1):
    c = lax.axis_index("c")
    s = lax.axis_index("s")

    n_init = n_nodes // cw

    def init_one(i, carry):
        ch = s + i * NS
        pltpu.sync_copy(hs_ref.at[c].at[pl.ds(ch * cw, cw)], gb0)
        pltpu.sync_copy(gb0, acc_sp.at[pl.ds(ch * cw, cw)])
        return carry

    my_n = (n_init - s + NS - 1) // NS
    lax.fori_loop(0, my_n, init_one, 0)
    plsc.subcore_barrier()

    def pair(m, carry):
        a = 2 * m
        g0 = pltpu.async_copy(hs_ref.at[c].at[ib.at[a].at[0]], gb0, gsem)
        g1 = pltpu.async_copy(hs_ref.at[c].at[ib.at[a + 1].at[0]], gb1, gsem)
        g0.wait()
        s0 = pltpu.async_copy(gb0, acc_sp.at[ib.at[a].at[1]], ssem, add=True)
        g1.wait()
        s1 = pltpu.async_copy(gb1, acc_sp.at[ib.at[a + 1].at[1]], ssem,
                              add=True)
        s0.wait()
        s1.wait()
        return carry

    off = 0
    for hn in half1:
        pltpu.sync_copy(idx_ref.at[s].at[pl.ds(off, hn)], ib.at[pl.ds(0, hn)])
        lax.fori_loop(0, hn // 2, pair, 0)
        off += hn

    plsc.subcore_barrier()

    def wb_one(i, carry):
        ch = s + i * NS
        pltpu.sync_copy(acc_sp.at[pl.ds(ch * cw, cw)], gb0)
        pltpu.sync_copy(gb0, acc_hbm.at[c].at[pl.ds(ch * cw, cw)])
        return carry

    lax.fori_loop(0, my_n, wb_one, 0)


def _gcn_scatter(hs, idx4):
    _, n, hf = hs.shape
    ns, nchunk, _, cw = idx4.shape
    assert ns == NS and n % cw == 0
    parts = [64] * (nchunk // 64)
    if nchunk % 64:
        parts.append(nchunk % 64)
    assert all(p % 2 == 0 for p in parts)
    mesh = plsc.VectorSubcoreMesh(core_axis_name="c", subcore_axis_name="s")
    k = pl.kernel(
        functools.partial(_scat_body, n_nodes=n, nchunk=nchunk, cw=cw,
                          half1=parts),
        out_type=jax.ShapeDtypeStruct((2, n, hf), jnp.float32),
        mesh=mesh,
        compiler_params=_SC_PARAMS,
        scratch_types=[
            pltpu.VMEM((64, 2, cw), jnp.int32),
            pltpu.VMEM((cw, hf), jnp.float32),
            pltpu.VMEM((cw, hf), jnp.float32),
            pltpu.SemaphoreType.DMA,
            pltpu.SemaphoreType.DMA,
            pltpu.VMEM_SHARED((n, hf), jnp.float32),
        ],
    )
    return k(hs, idx4)


def _pair_body(z_ref, idx_ref, zi_hbm, zj_hbm, ib, bi0, bi1, bj0, bj1,
               gsem, wsem, nchunk, cw):
    wid = lax.axis_index("s") * NC + lax.axis_index("c")
    pltpu.sync_copy(idx_ref.at[wid], ib)
    base = wid * nchunk * cw

    def do_chunk(a, bi, bj):
        gi = pltpu.async_copy(z_ref.at[ib.at[a].at[0]], bi, gsem)
        gj = pltpu.async_copy(z_ref.at[ib.at[a].at[1]], bj, gsem)
        gi.wait()
        wi = pltpu.async_copy(bi, zi_hbm.at[pl.ds(base + a * cw, cw)], wsem)
        gj.wait()
        wj = pltpu.async_copy(bj, zj_hbm.at[pl.ds(base + a * cw, cw)], wsem)
        return wi, wj

    def pair(m, carry):
        a = 2 * m
        gi0 = pltpu.async_copy(z_ref.at[ib.at[a].at[0]], bi0, gsem)
        gj0 = pltpu.async_copy(z_ref.at[ib.at[a].at[1]], bj0, gsem)
        gi1 = pltpu.async_copy(z_ref.at[ib.at[a + 1].at[0]], bi1, gsem)
        gj1 = pltpu.async_copy(z_ref.at[ib.at[a + 1].at[1]], bj1, gsem)
        gi0.wait()
        wi0 = pltpu.async_copy(bi0, zi_hbm.at[pl.ds(base + a * cw, cw)], wsem)
        gj0.wait()
        wj0 = pltpu.async_copy(bj0, zj_hbm.at[pl.ds(base + a * cw, cw)], wsem)
        gi1.wait()
        wi1 = pltpu.async_copy(bi1, zi_hbm.at[pl.ds(base + (a + 1) * cw, cw)],
                               wsem)
        gj1.wait()
        wj1 = pltpu.async_copy(bj1, zj_hbm.at[pl.ds(base + (a + 1) * cw, cw)],
                               wsem)
        wi0.wait()
        wj0.wait()
        wi1.wait()
        wj1.wait()
        return carry

    lax.fori_loop(0, nchunk // 2, pair, 0)
    if nchunk % 2:
        wi, wj = do_chunk(nchunk - 1, bi0, bj0)
        wi.wait()
        wj.wait()


def _pair_gather(z, idx4):
    n, h = z.shape
    nw, nchunk, _, cw = idx4.shape
    assert nw == NW
    e = nw * nchunk * cw
    mesh = plsc.VectorSubcoreMesh(core_axis_name="c", subcore_axis_name="s")
    k = pl.kernel(
        functools.partial(_pair_body, nchunk=nchunk, cw=cw),
        out_type=[
            jax.ShapeDtypeStruct((e, h), jnp.float32),
            jax.ShapeDtypeStruct((e, h), jnp.float32),
        ],
        mesh=mesh,
        compiler_params=_SC_PARAMS,
        scratch_types=[
            pltpu.VMEM((nchunk, 2, cw), jnp.int32),
            pltpu.VMEM((cw, h), jnp.float32),
            pltpu.VMEM((cw, h), jnp.float32),
            pltpu.VMEM((cw, h), jnp.float32),
            pltpu.VMEM((cw, h), jnp.float32),
            pltpu.SemaphoreType.DMA,
            pltpu.SemaphoreType.DMA,
        ],
    )
    return k(z, idx4)


def _t1_body(hist_ref, x_ref, w1_ref, hs_ref):
    deg = jnp.sum(hist_ref[...], axis=0) + 1.0
    dv = deg ** -0.5
    h = jnp.dot(x_ref[...], w1_ref[...], preferred_element_type=jnp.float32)
    hs = h * dv[:, None]
    hf = hs.shape[1] // 2
    hs_ref[0] = hs[:, :hf]
    hs_ref[1] = hs[:, hf:]


def _t1(hist, x, W1):
    n = x.shape[0]
    hid = W1.shape[1]
    return pl.pallas_call(
        _t1_body,
        out_shape=jax.ShapeDtypeStruct((2, n, hid // 2), jnp.float32),
    )(hist, x, W1)


def _t2_body(acc_ref, hist_ref, b1_ref, w2_ref, hs2_ref):
    deg = jnp.sum(hist_ref[...], axis=0) + 1.0
    dv = deg ** -0.5
    a0 = acc_ref[0] * dv[:, None]
    a1 = acc_ref[1] * dv[:, None]
    hf = a0.shape[1]
    z0 = jnp.maximum(a0 + b1_ref[0, :hf], 0.0)
    z1 = jnp.maximum(a1 + b1_ref[0, hf:], 0.0)
    h2 = (jnp.dot(z0, w2_ref[:hf], preferred_element_type=jnp.float32)
          + jnp.dot(z1, w2_ref[hf:], preferred_element_type=jnp.float32))
    hs2 = h2 * dv[:, None]
    hs2_ref[0] = hs2[:, :hf]
    hs2_ref[1] = hs2[:, hf:]


def _t2(acc, hist, b1, W2):
    _, n, hf = acc.shape
    return pl.pallas_call(
        _t2_body,
        out_shape=jax.ShapeDtypeStruct((2, n, hf), jnp.float32),
    )(acc, hist, b1, W2)


def _t3_body(acc_ref, hist_ref, b2_ref, z_ref):
    deg = jnp.sum(hist_ref[...], axis=0) + 1.0
    dv = deg ** -0.5
    hf = acc_ref.shape[2]
    z_ref[:, :hf] = acc_ref[0] * dv[:, None] + b2_ref[0, :hf]
    z_ref[:, hf:] = acc_ref[1] * dv[:, None] + b2_ref[0, hf:]


def _t3(acc, hist, b2):
    _, n, hf = acc.shape
    hid = 2 * hf
    return pl.pallas_call(
        _t3_body,
        out_shape=jax.ShapeDtypeStruct((n, hid), jnp.float32),
    )(acc, hist, b2)


def _t4_body(zi_ref, zj_ref, A_ref, B_ref, C_ref, bm1_ref, wm2_ref, o_ref):
    zi = zi_ref[...]
    zj = zj_ref[...]
    u = (jnp.dot(zi, A_ref[...], preferred_element_type=jnp.float32)
         + jnp.dot(zj, B_ref[...], preferred_element_type=jnp.float32)
         + jnp.dot(jnp.abs(zi - zj), C_ref[...],
                   preferred_element_type=jnp.float32)
         + bm1_ref[...])
    u = jnp.maximum(u, 0.0)
    o_ref[...] = jnp.dot(u, wm2_ref[...], preferred_element_type=jnp.float32)


def _t4(zi, zj, Wm1, bm1, Wm2, bt=2560):
    e, h = zi.shape
    assert e % bt == 0
    A = Wm1[0:h]
    B = Wm1[h:2 * h]
    C = Wm1[2 * h:3 * h]
    return pl.pallas_call(
        _t4_body,
        grid=(e // bt,),
        in_specs=[
            pl.BlockSpec((bt, h), lambda i: (i, 0)),
            pl.BlockSpec((bt, h), lambda i: (i, 0)),
            pl.BlockSpec((h, h), lambda i: (0, 0)),
            pl.BlockSpec((h, h), lambda i: (0, 0)),
            pl.BlockSpec((h, h), lambda i: (0, 0)),
            pl.BlockSpec((1, h), lambda i: (0, 0)),
            pl.BlockSpec((h, 1), lambda i: (0, 0)),
        ],
        out_specs=pl.BlockSpec((bt, 1), lambda i: (i, 0)),
        out_shape=jax.ShapeDtypeStruct((e, 1), jnp.float32),
    )(zi, zj, A, B, C, bm1.reshape(1, h), Wm2)


def kernel(x, edge_index, edge_label_index, W1, b1, W2, b2, Wm1, bm1, Wm2,
           bm2):
    n = x.shape[0]
    e = edge_index.shape[1]
    el = edge_label_index.shape[1]
    cw = 80
    ei = edge_index.astype(jnp.int32)
    eli = edge_label_index.astype(jnp.int32)
    idx_e = ei.reshape(2, NS, e // (NS * cw), cw).transpose(1, 2, 0, 3)
    idx_l = eli.reshape(2, NW, el // (NW * cw), cw).transpose(1, 2, 0, 3)

    hist = _degree_hist(ei[1], n)
    hs = _t1(hist, x, W1)
    acc = _gcn_scatter(hs, idx_e)
    hs2 = _t2(acc, hist, b1.reshape(1, -1), W2)
    acc2 = _gcn_scatter(hs2, idx_e)
    z = _t3(acc2, hist, b2.reshape(1, -1))
    zi, zj = _pair_gather(z, idx_l)
    out = _t4(zi, zj, Wm1, bm1, Wm2)
    return (out + bm2).squeeze()


# speedup vs baseline: 10.3208x; 1.0625x over previous
import functools

import jax
import jax.numpy as jnp
from jax import lax
from jax.experimental import pallas as pl
from jax.experimental.pallas import tpu as pltpu
from jax.experimental.pallas import tpu_sc as plsc

NC = 2
NS = 16
NW = NC * NS
LANES = 16

_SC_PARAMS = pltpu.CompilerParams(needs_layout_passes=False)


def _deg_body(dst_ref, hist_hbm, idx_v, hist_v, nvec):
    wid = lax.axis_index("s") * NC + lax.axis_index("c")
    pltpu.sync_copy(dst_ref.at[wid], idx_v)
    zero16 = jnp.zeros((LANES,), jnp.float32)

    def zbody(j, c):
        hist_v[pl.ds(j * LANES, LANES)] = zero16
        return c

    lax.fori_loop(0, hist_v.shape[0] // LANES, zbody, 0)

    lidx = lax.iota(jnp.int32, LANES)
    masks = [lidx == l for l in range(LANES)]
    ones = jnp.ones((LANES,), jnp.float32)

    def ebody(j, c):
        idx = idx_v[j]
        for l in range(LANES):
            plsc.addupdate_scatter(hist_v, [idx], ones, mask=masks[l])
        return c

    lax.fori_loop(0, nvec, ebody, 0)
    pltpu.sync_copy(hist_v, hist_hbm.at[wid])


def _degree_hist(dst, n_nodes):
    e = dst.size
    assert e % (NW * LANES) == 0
    nvec = e // (NW * LANES)
    dst3 = dst.reshape(NW, nvec, LANES)
    mesh = plsc.VectorSubcoreMesh(core_axis_name="c", subcore_axis_name="s")
    k = pl.kernel(
        functools.partial(_deg_body, nvec=nvec),
        out_type=jax.ShapeDtypeStruct((NW, n_nodes), jnp.float32),
        mesh=mesh,
        compiler_params=_SC_PARAMS,
        scratch_types=[
            pltpu.VMEM((nvec, LANES), jnp.int32),
            pltpu.VMEM((n_nodes,), jnp.float32),
        ],
    )
    return k(dst3)


def _scat_body(hs_ref, idx_ref, acc_hbm, ib, gb0, gb1, gsem, ssem,
               acc_sp, n_nodes, nchunk, cw, half1):
    c = lax.axis_index("c")
    s = lax.axis_index("s")

    n_init = n_nodes // cw

    def init_one(i, carry):
        ch = s + i * NS
        pltpu.sync_copy(hs_ref.at[c].at[pl.ds(ch * cw, cw)], gb0)
        pltpu.sync_copy(gb0, acc_sp.at[pl.ds(ch * cw, cw)])
        return carry

    my_n = (n_init - s + NS - 1) // NS
    lax.fori_loop(0, my_n, init_one, 0)
    plsc.subcore_barrier()

    def pair(m, carry):
        a = 2 * m
        g0 = pltpu.async_copy(hs_ref.at[c].at[ib.at[a].at[0]], gb0, gsem)
        g1 = pltpu.async_copy(hs_ref.at[c].at[ib.at[a + 1].at[0]], gb1, gsem)
        g0.wait()
        s0 = pltpu.async_copy(gb0, acc_sp.at[ib.at[a].at[1]], ssem, add=True)
        g1.wait()
        s1 = pltpu.async_copy(gb1, acc_sp.at[ib.at[a + 1].at[1]], ssem,
                              add=True)
        s0.wait()
        s1.wait()
        return carry

    off = 0
    for hn in half1:
        pltpu.sync_copy(idx_ref.at[s].at[pl.ds(off, hn)], ib.at[pl.ds(0, hn)])
        lax.fori_loop(0, hn // 2, pair, 0)
        off += hn

    plsc.subcore_barrier()

    def wb_one(i, carry):
        ch = s + i * NS
        pltpu.sync_copy(acc_sp.at[pl.ds(ch * cw, cw)], gb0)
        pltpu.sync_copy(gb0, acc_hbm.at[c].at[pl.ds(ch * cw, cw)])
        return carry

    lax.fori_loop(0, my_n, wb_one, 0)


def _gcn_scatter(hs, idx4):
    _, n, hf = hs.shape
    ns, nchunk, _, cw = idx4.shape
    assert ns == NS and n % cw == 0
    parts = [64] * (nchunk // 64)
    if nchunk % 64:
        parts.append(nchunk % 64)
    assert all(p % 2 == 0 for p in parts)
    mesh = plsc.VectorSubcoreMesh(core_axis_name="c", subcore_axis_name="s")
    k = pl.kernel(
        functools.partial(_scat_body, n_nodes=n, nchunk=nchunk, cw=cw,
                          half1=parts),
        out_type=jax.ShapeDtypeStruct((2, n, hf), jnp.float32),
        mesh=mesh,
        compiler_params=_SC_PARAMS,
        scratch_types=[
            pltpu.VMEM((64, 2, cw), jnp.int32),
            pltpu.VMEM((cw, hf), jnp.float32),
            pltpu.VMEM((cw, hf), jnp.float32),
            pltpu.SemaphoreType.DMA,
            pltpu.SemaphoreType.DMA,
            pltpu.VMEM_SHARED((n, hf), jnp.float32),
        ],
    )
    return k(hs, idx4)


_PG = 4


def _pair_body(z_ref, idx_ref, zi_hbm, zj_hbm, ib, *rest, nchunk, cw):
    bis = rest[:_PG]
    bjs = rest[_PG:2 * _PG]
    gsem, wsem = rest[2 * _PG:2 * _PG + 2]
    wid = lax.axis_index("s") * NC + lax.axis_index("c")
    pltpu.sync_copy(idx_ref.at[wid], ib)
    base = wid * nchunk * cw

    def group(m, carry):
        a = _PG * m
        gs = []
        for t in range(_PG):
            gs.append(pltpu.async_copy(z_ref.at[ib.at[a + t].at[0]], bis[t],
                                       gsem))
            gs.append(pltpu.async_copy(z_ref.at[ib.at[a + t].at[1]], bjs[t],
                                       gsem))
        ws = []
        for t in range(_PG):
            gs[2 * t].wait()
            ws.append(pltpu.async_copy(
                bis[t], zi_hbm.at[pl.ds(base + (a + t) * cw, cw)], wsem))
            gs[2 * t + 1].wait()
            ws.append(pltpu.async_copy(
                bjs[t], zj_hbm.at[pl.ds(base + (a + t) * cw, cw)], wsem))
        for w in ws:
            w.wait()
        return carry

    lax.fori_loop(0, nchunk // _PG, group, 0)
    for a in range(nchunk - nchunk % _PG, nchunk):
        t = a % _PG
        gi = pltpu.async_copy(z_ref.at[ib.at[a].at[0]], bis[t], gsem)
        gj = pltpu.async_copy(z_ref.at[ib.at[a].at[1]], bjs[t], gsem)
        gi.wait()
        wi = pltpu.async_copy(bis[t], zi_hbm.at[pl.ds(base + a * cw, cw)],
                              wsem)
        gj.wait()
        wj = pltpu.async_copy(bjs[t], zj_hbm.at[pl.ds(base + a * cw, cw)],
                              wsem)
        wi.wait()
        wj.wait()


def _pair_gather(zp, idx4):
    n, hw = zp.shape
    nw, nchunk, _, cw = idx4.shape
    assert nw == NW
    e = nw * nchunk * cw
    mesh = plsc.VectorSubcoreMesh(core_axis_name="c", subcore_axis_name="s")
    k = pl.kernel(
        functools.partial(_pair_body, nchunk=nchunk, cw=cw),
        out_type=[
            jax.ShapeDtypeStruct((e, hw), jnp.int32),
            jax.ShapeDtypeStruct((e, hw), jnp.int32),
        ],
        mesh=mesh,
        compiler_params=_SC_PARAMS,
        scratch_types=(
            [pltpu.VMEM((nchunk, 2, cw), jnp.int32)]
            + [pltpu.VMEM((cw, hw), jnp.int32) for _ in range(2 * _PG)]
            + [pltpu.SemaphoreType.DMA, pltpu.SemaphoreType.DMA]
        ),
    )
    return k(zp, idx4)


def _t1_body(hist_ref, x_ref, w1_ref, hs_ref):
    deg = jnp.sum(hist_ref[...], axis=0) + 1.0
    dv = deg ** -0.5
    h = jnp.dot(x_ref[...], w1_ref[...], preferred_element_type=jnp.float32)
    hs = h * dv[:, None]
    hf = hs.shape[1] // 2
    hs_ref[0] = hs[:, :hf]
    hs_ref[1] = hs[:, hf:]


def _t1(hist, x, W1):
    n = x.shape[0]
    hid = W1.shape[1]
    return pl.pallas_call(
        _t1_body,
        out_shape=jax.ShapeDtypeStruct((2, n, hid // 2), jnp.float32),
    )(hist, x, W1)


def _t2_body(acc_ref, hist_ref, b1_ref, w2_ref, hs2_ref):
    deg = jnp.sum(hist_ref[...], axis=0) + 1.0
    dv = deg ** -0.5
    a0 = acc_ref[0] * dv[:, None]
    a1 = acc_ref[1] * dv[:, None]
    hf = a0.shape[1]
    z0 = jnp.maximum(a0 + b1_ref[0, :hf], 0.0)
    z1 = jnp.maximum(a1 + b1_ref[0, hf:], 0.0)
    h2 = (jnp.dot(z0, w2_ref[:hf], preferred_element_type=jnp.float32)
          + jnp.dot(z1, w2_ref[hf:], preferred_element_type=jnp.float32))
    hs2 = h2 * dv[:, None]
    hs2_ref[0] = hs2[:, :hf]
    hs2_ref[1] = hs2[:, hf:]


def _t2(acc, hist, b1, W2):
    _, n, hf = acc.shape
    return pl.pallas_call(
        _t2_body,
        out_shape=jax.ShapeDtypeStruct((2, n, hf), jnp.float32),
    )(acc, hist, b1, W2)


def _t3_body(acc_ref, hist_ref, b2_ref, z_ref):
    deg = jnp.sum(hist_ref[...], axis=0) + 1.0
    dv = deg ** -0.5
    hf = acc_ref.shape[2]
    z_ref[:, :hf] = (acc_ref[0] * dv[:, None]
                     + b2_ref[0, :hf]).astype(jnp.bfloat16)
    z_ref[:, hf:] = (acc_ref[1] * dv[:, None]
                     + b2_ref[0, hf:]).astype(jnp.bfloat16)


def _t3(acc, hist, b2):
    _, n, hf = acc.shape
    hid = 2 * hf
    return pl.pallas_call(
        _t3_body,
        out_shape=jax.ShapeDtypeStruct((n, hid), jnp.bfloat16),
    )(acc, hist, b2)


def _unpack_bf16_pair(w):
    lo = lax.bitcast_convert_type(w << 16, jnp.float32)
    hi = lax.bitcast_convert_type(w & jnp.int32(-65536), jnp.float32)
    return lo.astype(jnp.bfloat16), hi.astype(jnp.bfloat16)


def _t4_body(zi_ref, zj_ref, Ae_ref, Ao_ref, Be_ref, Bo_ref, Ce_ref, Co_ref,
             bm1_ref, wm2_ref, o_ref):
    zie, zio = _unpack_bf16_pair(zi_ref[...])
    zje, zjo = _unpack_bf16_pair(zj_ref[...])
    f32 = jnp.float32
    u = (jnp.dot(zie, Ae_ref[...], preferred_element_type=f32)
         + jnp.dot(zio, Ao_ref[...], preferred_element_type=f32)
         + jnp.dot(zje, Be_ref[...], preferred_element_type=f32)
         + jnp.dot(zjo, Bo_ref[...], preferred_element_type=f32)
         + jnp.dot(jnp.abs(zie - zje), Ce_ref[...], preferred_element_type=f32)
         + jnp.dot(jnp.abs(zio - zjo), Co_ref[...], preferred_element_type=f32)
         + bm1_ref[...])
    u = jnp.maximum(u, 0.0)
    o_ref[...] = jnp.dot(u, wm2_ref[...], preferred_element_type=f32)


def _t4(zip_, zjp, Wm1, bm1, Wm2, bt=2560):
    e, hw = zip_.shape
    h = 2 * hw
    assert e % bt == 0
    Wb = Wm1.astype(jnp.bfloat16)
    A = Wb[0:h]
    B = Wb[h:2 * h]
    C = Wb[2 * h:3 * h]
    halves = [A[0::2], A[1::2], B[0::2], B[1::2], C[0::2], C[1::2]]
    wspec = [pl.BlockSpec((hw, h), lambda i: (0, 0)) for _ in range(6)]
    return pl.pallas_call(
        _t4_body,
        grid=(e // bt,),
        in_specs=[
            pl.BlockSpec((bt, hw), lambda i: (i, 0)),
            pl.BlockSpec((bt, hw), lambda i: (i, 0)),
        ] + wspec + [
            pl.BlockSpec((1, h), lambda i: (0, 0)),
            pl.BlockSpec((h, 1), lambda i: (0, 0)),
        ],
        out_specs=pl.BlockSpec((bt, 1), lambda i: (i, 0)),
        out_shape=jax.ShapeDtypeStruct((e, 1), jnp.float32),
    )(zip_, zjp, *halves, bm1.reshape(1, h), Wm2)


def kernel(x, edge_index, edge_label_index, W1, b1, W2, b2, Wm1, bm1, Wm2,
           bm2):
    n = x.shape[0]
    e = edge_index.shape[1]
    el = edge_label_index.shape[1]
    cw = 80
    ei = edge_index.astype(jnp.int32)
    eli = edge_label_index.astype(jnp.int32)
    idx_e = ei.reshape(2, NS, e // (NS * cw), cw).transpose(1, 2, 0, 3)
    idx_l = eli.reshape(2, NW, el // (NW * cw), cw).transpose(1, 2, 0, 3)

    hist = _degree_hist(ei[1], n)
    hs = _t1(hist, x, W1)
    acc = _gcn_scatter(hs, idx_e)
    hs2 = _t2(acc, hist, b1.reshape(1, -1), W2)
    acc2 = _gcn_scatter(hs2, idx_e)
    z = _t3(acc2, hist, b2.reshape(1, -1))
    h = z.shape[1]
    zp = lax.bitcast_convert_type(z.reshape(n, h // 2, 2), jnp.int32)
    zip_, zjp = _pair_gather(zp, idx_l)
    out = _t4(zip_, zjp, Wm1, bm1, Wm2)
    return (out + bm2).squeeze()


# speedup vs baseline: 10.5730x; 1.0244x over previous
import functools

import jax
import jax.numpy as jnp
from jax import lax
from jax.experimental import pallas as pl
from jax.experimental.pallas import tpu as pltpu
from jax.experimental.pallas import tpu_sc as plsc

NC = 2
NS = 16
NW = NC * NS
LANES = 16

_SC_PARAMS = pltpu.CompilerParams(needs_layout_passes=False)


def _deg_body(dst_ref, hist_hbm, idx_v, hist_v, nvec):
    wid = lax.axis_index("s") * NC + lax.axis_index("c")
    pltpu.sync_copy(dst_ref.at[wid], idx_v)
    zero16 = jnp.zeros((LANES,), jnp.float32)

    def zbody(j, c):
        hist_v[pl.ds(j * LANES, LANES)] = zero16
        return c

    lax.fori_loop(0, hist_v.shape[0] // LANES, zbody, 0)

    lidx = lax.iota(jnp.int32, LANES)
    masks = [lidx == l for l in range(LANES)]
    ones = jnp.ones((LANES,), jnp.float32)

    def ebody(j, c):
        idx = idx_v[j]
        for l in range(LANES):
            plsc.addupdate_scatter(hist_v, [idx], ones, mask=masks[l])
        return c

    lax.fori_loop(0, nvec, ebody, 0)
    pltpu.sync_copy(hist_v, hist_hbm.at[wid])


def _degree_hist(dst, n_nodes):
    e = dst.size
    assert e % (NW * LANES) == 0
    nvec = e // (NW * LANES)
    dst3 = dst.reshape(NW, nvec, LANES)
    mesh = plsc.VectorSubcoreMesh(core_axis_name="c", subcore_axis_name="s")
    k = pl.kernel(
        functools.partial(_deg_body, nvec=nvec),
        out_type=jax.ShapeDtypeStruct((NW, n_nodes), jnp.float32),
        mesh=mesh,
        compiler_params=_SC_PARAMS,
        scratch_types=[
            pltpu.VMEM((nvec, LANES), jnp.int32),
            pltpu.VMEM((n_nodes,), jnp.float32),
        ],
    )
    return k(dst3)


_SG = 3


def _scat_body(hs_ref, src_ref, dst_ref, acc_hbm, sv, dv, *rest,
               n_nodes, nchunk, cw, half1):
    gbufs = rest[:_SG]
    gsem, ssem, acc_sp = rest[_SG:_SG + 3]
    c = lax.axis_index("c")
    s = lax.axis_index("s")

    n_init = n_nodes // cw

    def init_one(i, carry):
        ch = s + i * NS
        pltpu.sync_copy(hs_ref.at[c].at[pl.ds(ch * cw, cw)], gbufs[0])
        pltpu.sync_copy(gbufs[0], acc_sp.at[pl.ds(ch * cw, cw)])
        return carry

    my_n = (n_init - s + NS - 1) // NS
    lax.fori_loop(0, my_n, init_one, 0)
    plsc.subcore_barrier()

    d = len(gbufs)

    def group(m, carry):
        a = d * m
        gs = [pltpu.async_copy(hs_ref.at[c].at[sv.at[a + t]], gbufs[t], gsem)
              for t in range(d)]
        ss = []
        for t in range(d):
            gs[t].wait()
            ss.append(pltpu.async_copy(gbufs[t], acc_sp.at[dv.at[a + t]],
                                       ssem, add=True))
        for sc in ss:
            sc.wait()
        return carry

    off = 0
    for hn, proc in half1:
        pltpu.sync_copy(src_ref.at[s].at[pl.ds(off, hn)], sv.at[pl.ds(0, hn)])
        pltpu.sync_copy(dst_ref.at[s].at[pl.ds(off, hn)], dv.at[pl.ds(0, hn)])
        lax.fori_loop(0, proc // d, group, 0)
        for a in range(proc - proc % d, proc):
            g = pltpu.async_copy(hs_ref.at[c].at[sv.at[a]], gbufs[0], gsem)
            g.wait()
            sc = pltpu.async_copy(gbufs[0], acc_sp.at[dv.at[a]], ssem,
                                  add=True)
            sc.wait()
        off += hn

    plsc.subcore_barrier()

    def wb_one(i, carry):
        ch = s + i * NS
        pltpu.sync_copy(acc_sp.at[pl.ds(ch * cw, cw)], gbufs[0])
        pltpu.sync_copy(gbufs[0], acc_hbm.at[c].at[pl.ds(ch * cw, cw)])
        return carry

    lax.fori_loop(0, my_n, wb_one, 0)


_SP = 24


def _gcn_scatter(hs, src3, dst3, nreal):
    _, n, hf = hs.shape
    ns, nchunk, cw = src3.shape
    assert ns == NS and n % cw == 0
    parts = []
    off = 0
    while off < nreal:
        stage = min(_SP, nchunk - off)
        parts.append((stage, min(_SP, nreal - off)))
        off += stage
    mesh = plsc.VectorSubcoreMesh(core_axis_name="c", subcore_axis_name="s")
    k = pl.kernel(
        functools.partial(_scat_body, n_nodes=n, nchunk=nchunk, cw=cw,
                          half1=parts),
        out_type=jax.ShapeDtypeStruct((2, n, hf), jnp.float32),
        mesh=mesh,
        compiler_params=_SC_PARAMS,
        scratch_types=(
            [pltpu.VMEM((_SP, cw), jnp.int32),
             pltpu.VMEM((_SP, cw), jnp.int32)]
            + [pltpu.VMEM((cw, hf), jnp.float32) for _ in range(_SG)]
            + [pltpu.SemaphoreType.DMA, pltpu.SemaphoreType.DMA,
               pltpu.VMEM_SHARED((n, hf), jnp.float32)]
        ),
    )
    return k(hs, src3, dst3)


def _pad_chunks(a, m):
    pad = (-a.shape[1]) % m
    if pad:
        a = jnp.pad(a, ((0, 0), (0, pad), (0, 0)))
    return a


_PG = 4


def _pair_body(z_ref, ii_ref, jj_ref, zi_hbm, zj_hbm, iv, jv, *rest,
               nchunk, cw):
    bis = rest[:_PG]
    bjs = rest[_PG:2 * _PG]
    gsem, wsem = rest[2 * _PG:2 * _PG + 2]
    wid = lax.axis_index("s") * NC + lax.axis_index("c")
    pltpu.sync_copy(ii_ref.at[wid], iv)
    pltpu.sync_copy(jj_ref.at[wid], jv)
    base = wid * nchunk * cw

    def group(m, carry):
        a = _PG * m
        gs = []
        for t in range(_PG):
            gs.append(pltpu.async_copy(z_ref.at[iv.at[a + t]], bis[t],
                                       gsem))
            gs.append(pltpu.async_copy(z_ref.at[jv.at[a + t]], bjs[t],
                                       gsem))
        ws = []
        for t in range(_PG):
            gs[2 * t].wait()
            ws.append(pltpu.async_copy(
                bis[t], zi_hbm.at[pl.ds(base + (a + t) * cw, cw)], wsem))
            gs[2 * t + 1].wait()
            ws.append(pltpu.async_copy(
                bjs[t], zj_hbm.at[pl.ds(base + (a + t) * cw, cw)], wsem))
        for w in ws:
            w.wait()
        return carry

    lax.fori_loop(0, nchunk // _PG, group, 0)
    for a in range(nchunk - nchunk % _PG, nchunk):
        t = a % _PG
        gi = pltpu.async_copy(z_ref.at[iv.at[a]], bis[t], gsem)
        gj = pltpu.async_copy(z_ref.at[jv.at[a]], bjs[t], gsem)
        gi.wait()
        wi = pltpu.async_copy(bis[t], zi_hbm.at[pl.ds(base + a * cw, cw)],
                              wsem)
        gj.wait()
        wj = pltpu.async_copy(bjs[t], zj_hbm.at[pl.ds(base + a * cw, cw)],
                              wsem)
        wi.wait()
        wj.wait()


def _pair_gather(zp, ii3, jj3):
    n, hw = zp.shape
    nw, nchunk, cw = ii3.shape
    assert nw == NW
    e = nw * nchunk * cw
    mesh = plsc.VectorSubcoreMesh(core_axis_name="c", subcore_axis_name="s")
    k = pl.kernel(
        functools.partial(_pair_body, nchunk=nchunk, cw=cw),
        out_type=[
            jax.ShapeDtypeStruct((e, hw), jnp.int32),
            jax.ShapeDtypeStruct((e, hw), jnp.int32),
        ],
        mesh=mesh,
        compiler_params=_SC_PARAMS,
        scratch_types=(
            [pltpu.VMEM((nchunk, cw), jnp.int32),
             pltpu.VMEM((nchunk, cw), jnp.int32)]
            + [pltpu.VMEM((cw, hw), jnp.int32) for _ in range(2 * _PG)]
            + [pltpu.SemaphoreType.DMA, pltpu.SemaphoreType.DMA]
        ),
    )
    return k(zp, ii3, jj3)


def _t1_body(hist_ref, x_ref, w1_ref, hs_ref):
    deg = jnp.sum(hist_ref[...], axis=0) + 1.0
    dv = deg ** -0.5
    h = jnp.dot(x_ref[...], w1_ref[...], preferred_element_type=jnp.float32)
    hs = h * dv[:, None]
    hf = hs.shape[1] // 2
    hs_ref[0] = hs[:, :hf]
    hs_ref[1] = hs[:, hf:]


def _t1(hist, x, W1):
    n = x.shape[0]
    hid = W1.shape[1]
    return pl.pallas_call(
        _t1_body,
        out_shape=jax.ShapeDtypeStruct((2, n, hid // 2), jnp.float32),
    )(hist, x, W1)


def _t2_body(acc_ref, hist_ref, b1_ref, w2_ref, hs2_ref):
    deg = jnp.sum(hist_ref[...], axis=0) + 1.0
    dv = deg ** -0.5
    a0 = acc_ref[0] * dv[:, None]
    a1 = acc_ref[1] * dv[:, None]
    hf = a0.shape[1]
    z0 = jnp.maximum(a0 + b1_ref[0, :hf], 0.0)
    z1 = jnp.maximum(a1 + b1_ref[0, hf:], 0.0)
    h2 = (jnp.dot(z0, w2_ref[:hf], preferred_element_type=jnp.float32)
          + jnp.dot(z1, w2_ref[hf:], preferred_element_type=jnp.float32))
    hs2 = h2 * dv[:, None]
    hs2_ref[0] = hs2[:, :hf]
    hs2_ref[1] = hs2[:, hf:]


def _t2(acc, hist, b1, W2):
    _, n, hf = acc.shape
    return pl.pallas_call(
        _t2_body,
        out_shape=jax.ShapeDtypeStruct((2, n, hf), jnp.float32),
    )(acc, hist, b1, W2)


def _t3_body(acc_ref, hist_ref, b2_ref, z_ref):
    deg = jnp.sum(hist_ref[...], axis=0) + 1.0
    dv = deg ** -0.5
    hf = acc_ref.shape[2]
    z_ref[:, :hf] = (acc_ref[0] * dv[:, None]
                     + b2_ref[0, :hf]).astype(jnp.bfloat16)
    z_ref[:, hf:] = (acc_ref[1] * dv[:, None]
                     + b2_ref[0, hf:]).astype(jnp.bfloat16)


def _t3(acc, hist, b2):
    _, n, hf = acc.shape
    hid = 2 * hf
    return pl.pallas_call(
        _t3_body,
        out_shape=jax.ShapeDtypeStruct((n, hid), jnp.bfloat16),
    )(acc, hist, b2)


def _unpack_bf16_pair(w):
    lo = lax.bitcast_convert_type(w << 16, jnp.float32)
    hi = lax.bitcast_convert_type(w & jnp.int32(-65536), jnp.float32)
    return lo.astype(jnp.bfloat16), hi.astype(jnp.bfloat16)


def _t4_body(zi_ref, zj_ref, Ae_ref, Ao_ref, Be_ref, Bo_ref, Ce_ref, Co_ref,
             bm1_ref, wm2_ref, o_ref):
    zie, zio = _unpack_bf16_pair(zi_ref[...])
    zje, zjo = _unpack_bf16_pair(zj_ref[...])
    f32 = jnp.float32
    u = (jnp.dot(zie, Ae_ref[...], preferred_element_type=f32)
         + jnp.dot(zio, Ao_ref[...], preferred_element_type=f32)
         + jnp.dot(zje, Be_ref[...], preferred_element_type=f32)
         + jnp.dot(zjo, Bo_ref[...], preferred_element_type=f32)
         + jnp.dot(jnp.abs(zie - zje), Ce_ref[...], preferred_element_type=f32)
         + jnp.dot(jnp.abs(zio - zjo), Co_ref[...], preferred_element_type=f32)
         + bm1_ref[...])
    u = jnp.maximum(u, 0.0)
    o_ref[...] = jnp.dot(u, wm2_ref[...], preferred_element_type=f32)


def _t4(zip_, zjp, Wm1, bm1, Wm2, bt=2560):
    e, hw = zip_.shape
    h = 2 * hw
    assert e % bt == 0
    Wb = Wm1.astype(jnp.bfloat16)
    A = Wb[0:h]
    B = Wb[h:2 * h]
    C = Wb[2 * h:3 * h]
    halves = [A[0::2], A[1::2], B[0::2], B[1::2], C[0::2], C[1::2]]
    wspec = [pl.BlockSpec((hw, h), lambda i: (0, 0)) for _ in range(6)]
    return pl.pallas_call(
        _t4_body,
        grid=(e // bt,),
        in_specs=[
            pl.BlockSpec((bt, hw), lambda i: (i, 0)),
            pl.BlockSpec((bt, hw), lambda i: (i, 0)),
        ] + wspec + [
            pl.BlockSpec((1, h), lambda i: (0, 0)),
            pl.BlockSpec((h, 1), lambda i: (0, 0)),
        ],
        out_specs=pl.BlockSpec((bt, 1), lambda i: (i, 0)),
        out_shape=jax.ShapeDtypeStruct((e, 1), jnp.float32),
    )(zip_, zjp, *halves, bm1.reshape(1, h), Wm2)


def kernel(x, edge_index, edge_label_index, W1, b1, W2, b2, Wm1, bm1, Wm2,
           bm2):
    n = x.shape[0]
    e = edge_index.shape[1]
    el = edge_label_index.shape[1]
    cw = 80
    ei = edge_index.astype(jnp.int32)
    eli = edge_label_index.astype(jnp.int32)
    ne = e // (NS * cw)
    src3 = _pad_chunks(ei[0].reshape(NS, ne, cw), 8)
    dst3 = _pad_chunks(ei[1].reshape(NS, ne, cw), 8)
    ii3 = eli[0].reshape(NW, el // (NW * cw), cw)
    jj3 = eli[1].reshape(NW, el // (NW * cw), cw)

    hist = _degree_hist(ei[1], n)
    hs = _t1(hist, x, W1)
    acc = _gcn_scatter(hs, src3, dst3, ne)
    hs2 = _t2(acc, hist, b1.reshape(1, -1), W2)
    acc2 = _gcn_scatter(hs2, src3, dst3, ne)
    z = _t3(acc2, hist, b2.reshape(1, -1))
    h = z.shape[1]
    zp = lax.bitcast_convert_type(z.reshape(n, h // 2, 2), jnp.int32)
    zip_, zjp = _pair_gather(zp, ii3, jj3)
    out = _t4(zip_, zjp, Wm1, bm1, Wm2)
    return (out + bm2).squeeze()
